# baseline (speedup 1.0000x reference)
import functools
import math

import jax
import jax.numpy as jnp
from jax import lax
from jax.experimental import pallas as pl
from jax.experimental.pallas import tpu as pltpu
from jax.experimental.pallas import tpu_sc as plsc

B, M, D = 1024, 65536, 256
TM = 2048
NBLK = M // TM
SCALE = 1.0 / math.sqrt(D)


def _pass1_body(episode, Wq, bq, Wk, bk, mem_blk, scores_ref, m_ref, l_ref, q_s):
    i = pl.program_id(0)

    @pl.when(i == 0)
    def _init():
        q = lax.dot_general(episode[...], Wq[...], (((1,), (1,)), ((), ())),
                            preferred_element_type=jnp.float32)
        q_s[...] = (q + bq[...][None, :]) * SCALE
        m_ref[...] = jnp.full((B, 1), -jnp.inf, jnp.float32)
        l_ref[...] = jnp.zeros((B, 1), jnp.float32)

    k = lax.dot_general(mem_blk[...], Wk[...], (((1,), (1,)), ((), ())),
                        preferred_element_type=jnp.float32) + bk[...][None, :]
    s = lax.dot_general(q_s[...], k, (((1,), (1,)), ((), ())),
                        preferred_element_type=jnp.float32)
    scores_ref[...] = s
    bm = jnp.max(s, axis=1, keepdims=True)
    m_old = m_ref[...]
    m_new = jnp.maximum(m_old, bm)
    l_ref[...] = l_ref[...] * jnp.exp(m_old - m_new) + jnp.sum(
        jnp.exp(s - m_new), axis=1, keepdims=True)
    m_ref[...] = m_new


def _pass2_body(scores, mem_blk, m, l, Wv, bv, usage_blk, add1_blk,
                attn_ref, retr_ref, usage_out_ref, racc):
    i = pl.program_id(0)

    @pl.when(i == 0)
    def _init():
        racc[...] = jnp.zeros((B, D), jnp.float32)

    p = jnp.exp(scores[...] - m[...]) * (1.0 / l[...])
    attn_ref[...] = p
    v = lax.dot_general(mem_blk[...], Wv[...], (((1,), (1,)), ((), ())),
                        preferred_element_type=jnp.float32) + bv[...][None, :]
    racc[...] += lax.dot_general(p, v, (((1,), (0,)), ((), ())),
                                 preferred_element_type=jnp.float32)
    usage_out_ref[...] = usage_blk[...] + add1_blk[...] + jnp.sum(
        p, axis=0, keepdims=True)[None]

    @pl.when(i == NBLK - 1)
    def _fin():
        retr_ref[...] = racc[...]


def _attention(episode, memory, usage2, add1_2, Wq, bq, Wk, bk, Wv, bv):
    scores, m, l = pl.pallas_call(
        _pass1_body,
        grid=(NBLK,),
        in_specs=[
            pl.BlockSpec((B, D), lambda i: (0, 0)),
            pl.BlockSpec((D, D), lambda i: (0, 0)),
            pl.BlockSpec((D,), lambda i: (0,)),
            pl.BlockSpec((D, D), lambda i: (0, 0)),
            pl.BlockSpec((D,), lambda i: (0,)),
            pl.BlockSpec((TM, D), lambda i: (i, 0)),
        ],
        out_specs=[
            pl.BlockSpec((B, TM), lambda i: (0, i)),
            pl.BlockSpec((B, 1), lambda i: (0, 0)),
            pl.BlockSpec((B, 1), lambda i: (0, 0)),
        ],
        out_shape=[
            jax.ShapeDtypeStruct((B, M), jnp.float32),
            jax.ShapeDtypeStruct((B, 1), jnp.float32),
            jax.ShapeDtypeStruct((B, 1), jnp.float32),
        ],
        scratch_shapes=[pltpu.VMEM((B, D), jnp.float32)],
    )(episode, Wq, bq, Wk, bk, memory)

    attn, retrieved, usage_out = pl.pallas_call(
        _pass2_body,
        grid=(NBLK,),
        in_specs=[
            pl.BlockSpec((B, TM), lambda i: (0, i)),
            pl.BlockSpec((TM, D), lambda i: (i, 0)),
            pl.BlockSpec((B, 1), lambda i: (0, 0)),
            pl.BlockSpec((B, 1), lambda i: (0, 0)),
            pl.BlockSpec((D, D), lambda i: (0, 0)),
            pl.BlockSpec((D,), lambda i: (0,)),
            pl.BlockSpec((1, 1, TM), lambda i: (i, 0, 0)),
            pl.BlockSpec((1, 1, TM), lambda i: (i, 0, 0)),
        ],
        out_specs=[
            pl.BlockSpec((B, TM), lambda i: (0, i)),
            pl.BlockSpec((B, D), lambda i: (0, 0)),
            pl.BlockSpec((1, 1, TM), lambda i: (i, 0, 0)),
        ],
        out_shape=[
            jax.ShapeDtypeStruct((B, M), jnp.float32),
            jax.ShapeDtypeStruct((B, D), jnp.float32),
            jax.ShapeDtypeStruct((NBLK, 1, TM), jnp.float32),
        ],
        scratch_shapes=[pltpu.VMEM((B, D), jnp.float32)],
        input_output_aliases={0: 0},
    )(scores, memory, m, l, Wv, bv, usage2, add1_2)
    return attn, retrieved, usage_out


SC_NT = 16
SC_NLOC = M // SC_NT
SC_NV = SC_NLOC // 16
SC_ES = B // SC_NT
_MSB = -2147483648


def _sck1_body(age_hbm, lru_hbm, add1_hbm, maxage_hbm,
               age_v, key_v, selm_v, hist2d, hist_loc, ghist, rowtmp, maxtmp,
               selg_v, selk_v, scat_idx, ggidx_v, gkey_v, myrank_v, myg_v,
               stagef, stagei,
               sh_hist, sh_cnt, sh_max, gsel_gidx, gsel_key):
    sid = lax.axis_index("s")
    base = sid * SC_NLOC
    iota = lax.iota(jnp.int32, 16)
    ones_i = jnp.ones((16,), jnp.int32)

    pltpu.sync_copy(age_hbm.at[pl.ds(base, SC_NLOC)], age_v)

    def _keys(i, mx):
        a = age_v[pl.ds(i * 16, 16)]
        b = plsc.bitcast(a, jnp.int32)
        key_v[pl.ds(i * 16, 16)] = jnp.where(
            b >= 0, b, b ^ jnp.int32(0x7FFFFFFF))
        return jnp.maximum(mx, a)

    mx = lax.fori_loop(0, SC_NV, _keys, jnp.full((16,), -jnp.inf, jnp.float32))
    stagef[...] = mx
    pltpu.sync_copy(stagef, sh_max.at[pl.ds(sid * 16, 16)])

    def _round(digit_fn, cand_fn, krem):
        def _zero(i, _):
            hist2d[pl.ds(i * 16, 16)] = jnp.zeros((16,), jnp.int32)
            return 0
        lax.fori_loop(0, SC_NV, _zero, 0)

        def _acc(i, _):
            key = key_v[pl.ds(i * 16, 16)]
            ub = key ^ _MSB
            gidx = base + i * 16 + iota
            plsc.addupdate_scatter(hist2d, [iota * 256 + digit_fn(ub, gidx)],
                                   ones_i, mask=cand_fn(ub, gidx))
            return 0
        lax.fori_loop(0, SC_NV, _acc, 0)

        def _drain(j, _):
            def _lane(l, acc):
                return acc + hist2d[pl.ds(l * 256 + j * 16, 16)]
            hist_loc[pl.ds(j * 16, 16)] = lax.fori_loop(
                0, 16, _lane, jnp.zeros((16,), jnp.int32))
            return 0
        lax.fori_loop(0, 16, _drain, 0)

        pltpu.sync_copy(hist_loc, sh_hist.at[pl.ds(sid * 256, 256)])
        plsc.subcore_barrier()

        def _gz(i, _):
            ghist[pl.ds(i * 16, 16)] = jnp.zeros((16,), jnp.int32)
            return 0
        lax.fori_loop(0, 16, _gz, 0)

        def _comb(t, _):
            pltpu.sync_copy(sh_hist.at[pl.ds(t * 256, 256)], rowtmp)

            def _addv(i, _):
                ghist[pl.ds(i * 16, 16)] += rowtmp[pl.ds(i * 16, 16)]
                return 0
            lax.fori_loop(0, 16, _addv, 0)
            return 0
        lax.fori_loop(0, SC_NT, _comb, 0)
        plsc.subcore_barrier()

        def _find(i, carry):
            csum, dstar = carry
            pc = plsc.cumsum(ghist[pl.ds(i * 16, 16)]) + csum
            cand_d = jnp.where(pc >= krem, i * 16 + iota, jnp.int32(300))
            return jnp.max(pc), jnp.minimum(dstar, jnp.min(cand_d))

        _, dstar = lax.fori_loop(0, 16, _find, (jnp.int32(0), jnp.int32(300)))

        def _cl(i, c):
            v = ghist[pl.ds(i * 16, 16)]
            return c + jnp.sum(jnp.where(i * 16 + iota < dstar, v,
                                         jnp.int32(0)))
        cnt_lt = lax.fori_loop(0, 16, _cl, jnp.int32(0))
        return dstar, krem - cnt_lt

    krem = jnp.int32(B)
    d0, krem = _round(
        lambda ub, g: lax.shift_right_logical(ub, 24),
        lambda ub, g: jnp.ones((16,), jnp.bool_), krem)
    p = d0
    d1, krem = _round(
        lambda ub, g: lax.shift_right_logical(ub, 16) & 255,
        lambda ub, g: lax.shift_right_logical(ub, 24) == p, krem)
    p = (p << 8) | d1
    d2, krem = _round(
        lambda ub, g: lax.shift_right_logical(ub, 8) & 255,
        lambda ub, g: lax.shift_right_logical(ub, 16) == p, krem)
    p = (p << 8) | d2
    d3, krem = _round(
        lambda ub, g: ub & 255,
        lambda ub, g: lax.shift_right_logical(ub, 8) == p, krem)
    tub = (p << 8) | d3
    e0, krem = _round(
        lambda ub, g: lax.shift_right_logical(g, 8),
        lambda ub, g: ub == tub, krem)
    q = e0
    e1, krem = _round(
        lambda ub, g: g & 255,
        lambda ub, g: (ub == tub) & (lax.shift_right_logical(g, 8) == q),
        krem)
    tgidx = (q << 8) | e1
    tkey = tub ^ _MSB

    def _sel(i, cnt):
        key = key_v[pl.ds(i * 16, 16)]
        gidx = base + i * 16 + iota
        sel = (key < tkey) | ((key == tkey) & (gidx <= tgidx))
        selm_v[pl.ds(i * 16, 16)] = jnp.where(sel, 1.0, 0.0).astype(
            jnp.float32)
        cs = plsc.cumsum(sel.astype(jnp.int32))
        pos = cnt + cs - 1
        plsc.store_scatter(selg_v, [pos], gidx, mask=sel)
        plsc.store_scatter(selk_v, [pos], key, mask=sel)
        return cnt + jnp.max(cs)

    cnt = lax.fori_loop(0, SC_NV, _sel, jnp.int32(0))
    pltpu.sync_copy(selm_v, add1_hbm.at[pl.ds(base, SC_NLOC)])

    stagei[...] = jnp.full((16,), cnt, jnp.int32)
    pltpu.sync_copy(stagei, sh_cnt.at[pl.ds(sid * 16, 16)])
    plsc.subcore_barrier()
    pltpu.sync_copy(sh_cnt, rowtmp)

    def _off(t, o):
        c_t = rowtmp[pl.ds(t * 16, 16)][0]
        return o + jnp.where(t < sid, c_t, jnp.int32(0))
    off = lax.fori_loop(0, SC_NT, _off, jnp.int32(0))

    def _fillidx(i, _):
        posl = i * 16 + iota
        idxv = jnp.where(posl < cnt, off + posl, B + (posl & 127))
        plsc.store_scatter(scat_idx,
                           [lax.shift_right_logical(posl, 7), posl & 127],
                           idxv)
        return 0
    lax.fori_loop(0, SC_NV, _fillidx, 0)

    def _push(c, _):
        @pl.when(c * 128 < cnt)
        def _():
            pltpu.sync_copy(selg_v.at[pl.ds(c * 128, 128)],
                            gsel_gidx.at[scat_idx.at[c]])
            pltpu.sync_copy(selk_v.at[pl.ds(c * 128, 128)],
                            gsel_key.at[scat_idx.at[c]])
        return 0
    lax.fori_loop(0, SC_NLOC // 128, _push, 0)
    plsc.subcore_barrier()

    pltpu.sync_copy(gsel_gidx.at[pl.ds(0, B)], ggidx_v)
    pltpu.sync_copy(gsel_key.at[pl.ds(0, B)], gkey_v)

    def _rankgrp(j, _):
        ek = gkey_v[pl.ds(sid * SC_ES + j * 16, 16)]
        eg = ggidx_v[pl.ds(sid * SC_ES + j * 16, 16)]

        def _cand(i, acc):
            ckv = gkey_v[pl.ds(i * 16, 16)]
            cgv = ggidx_v[pl.ds(i * 16, 16)]
            for l in range(16):
                ck = ckv[l]
                cg = cgv[l]
                less = (ck < ek) | ((ck == ek) & (cg < eg))
                acc = acc + less.astype(jnp.int32)
            return acc

        accv = lax.fori_loop(0, B // 16, _cand, jnp.zeros((16,), jnp.int32))
        myrank_v[pl.ds(j * 16, 16)] = accv
        myg_v[pl.ds(j * 16, 16)] = eg
        return 0
    lax.fori_loop(0, SC_ES // 16, _rankgrp, 0)

    pltpu.sync_copy(myg_v, lru_hbm.at[myrank_v])

    @pl.when(sid == 0)
    def _wmax():
        pltpu.sync_copy(sh_max, maxtmp)

        def _gm(i, m):
            return jnp.maximum(m, maxtmp[pl.ds(i * 16, 16)])
        gmv = lax.fori_loop(0, SC_NT, _gm,
                            jnp.full((16,), -jnp.inf, jnp.float32))
        stagef[...] = jnp.full((16,), jnp.max(gmv), jnp.float32)
        pltpu.sync_copy(stagef, maxage_hbm)


def _sc_topk(memory_age):
    mesh = plsc.VectorSubcoreMesh(core_axis_name="c", subcore_axis_name="s",
                                  num_cores=1, num_subcores=16)
    f = pl.kernel(
        _sck1_body,
        out_type=[
            jax.ShapeDtypeStruct((B,), jnp.int32),
            jax.ShapeDtypeStruct((M,), jnp.float32),
            jax.ShapeDtypeStruct((16,), jnp.float32),
        ],
        mesh=mesh,
        compiler_params=pltpu.CompilerParams(needs_layout_passes=False),
        scratch_types=[
            pltpu.VMEM((SC_NLOC,), jnp.float32),
            pltpu.VMEM((SC_NLOC,), jnp.int32),
            pltpu.VMEM((SC_NLOC,), jnp.float32),
            pltpu.VMEM((SC_NLOC,), jnp.int32),
            pltpu.VMEM((256,), jnp.int32),
            pltpu.VMEM((256,), jnp.int32),
            pltpu.VMEM((256,), jnp.int32),
            pltpu.VMEM((256,), jnp.float32),
            pltpu.VMEM((SC_NLOC,), jnp.int32),
            pltpu.VMEM((SC_NLOC,), jnp.int32),
            pltpu.VMEM((SC_NLOC // 128, 128), jnp.int32),
            pltpu.VMEM((B,), jnp.int32),
            pltpu.VMEM((B,), jnp.int32),
            pltpu.VMEM((SC_ES,), jnp.int32),
            pltpu.VMEM((SC_ES,), jnp.int32),
            pltpu.VMEM((16,), jnp.float32),
            pltpu.VMEM((16,), jnp.int32),
            pltpu.VMEM_SHARED((SC_NT * 256,), jnp.int32),
            pltpu.VMEM_SHARED((SC_NT * 16,), jnp.int32),
            pltpu.VMEM_SHARED((SC_NT * 16,), jnp.float32),
            pltpu.VMEM_SHARED((B + 128,), jnp.int32),
            pltpu.VMEM_SHARED((B + 128,), jnp.int32),
        ],
    )
    return f(memory_age)


A_NT = 32
A_ROWS = M // A_NT


def _sck2_body(mem_hbm, ep_hbm, age_hbm, lru_hbm, maxage_hbm,
               newmem_hbm, newage_hbm,
               lru_v, rbuf, dbuf, ebuf, valb, mageb, sem):
    cid = lax.axis_index("c")
    sid = lax.axis_index("s")
    wid = sid * 2 + cid
    base = wid * A_ROWS
    iota = lax.iota(jnp.int32, 16)

    cp = pltpu.async_copy(mem_hbm.at[pl.ds(base, A_ROWS)],
                          newmem_hbm.at[pl.ds(base, A_ROWS)], sem)
    pltpu.sync_copy(age_hbm.at[pl.ds(base, A_ROWS)],
                    newage_hbm.at[pl.ds(base, A_ROWS)])
    pltpu.sync_copy(lru_hbm, lru_v)
    pltpu.sync_copy(maxage_hbm, mageb)
    newv = mageb[...] + 1.0
    valb[pl.ds(0, 16)] = newv
    valb[pl.ds(16, 16)] = newv

    def _scan(i, cnt):
        li = lru_v[pl.ds(i * 16, 16)]
        m = (li >= base) & (li < base + A_ROWS)
        cs = plsc.cumsum(m.astype(jnp.int32))
        pos = cnt + cs - 1
        plsc.store_scatter(rbuf, [lax.shift_right_logical(pos, 5), pos & 31],
                           i * 16 + iota, mask=m)
        plsc.store_scatter(dbuf, [lax.shift_right_logical(pos, 5), pos & 31],
                           li, mask=m)
        return cnt + jnp.max(cs)

    cnt = lax.fori_loop(0, B // 16, _scan, jnp.int32(0))

    @pl.when(cnt > 0)
    def _pad():
        last = cnt - 1
        lr = jnp.full((16,), lax.shift_right_logical(last, 5), jnp.int32)
        lc = jnp.full((16,), last & 31, jnp.int32)
        last_r = plsc.load_gather(rbuf, [lr, lc])
        last_d = plsc.load_gather(dbuf, [lr, lc])

        def _fill(i, _):
            posl = i * 16 + iota
            m = posl >= cnt
            plsc.store_scatter(rbuf,
                               [lax.shift_right_logical(posl, 5), posl & 31],
                               last_r, mask=m)
            plsc.store_scatter(dbuf,
                               [lax.shift_right_logical(posl, 5), posl & 31],
                               last_d, mask=m)
            return 0
        lax.fori_loop(0, B // 16, _fill, 0)

    cp.wait()

    def _chunk(c, _):
        @pl.when(c * 32 < cnt)
        def _():
            pltpu.async_copy(ep_hbm.at[rbuf.at[c]], ebuf, sem).wait()
            pltpu.async_copy(ebuf, newmem_hbm.at[dbuf.at[c]], sem).wait()
            pltpu.sync_copy(valb, newage_hbm.at[dbuf.at[c]])
        return 0
    lax.fori_loop(0, B // 32, _chunk, 0)


def _sc_apply(memory, episode, memory_age, lru, maxage):
    mesh = plsc.VectorSubcoreMesh(core_axis_name="c", subcore_axis_name="s",
                                  num_cores=2, num_subcores=16)
    f = pl.kernel(
        _sck2_body,
        out_type=[
            jax.ShapeDtypeStruct((M, D), jnp.float32),
            jax.ShapeDtypeStruct((M,), jnp.float32),
        ],
        mesh=mesh,
        compiler_params=pltpu.CompilerParams(needs_layout_passes=False),
        scratch_types=[
            pltpu.VMEM((B,), jnp.int32),
            pltpu.VMEM((B // 32, 32), jnp.int32),
            pltpu.VMEM((B // 32, 32), jnp.int32),
            pltpu.VMEM((32, D), jnp.float32),
            pltpu.VMEM((32,), jnp.float32),
            pltpu.VMEM((16,), jnp.float32),
            pltpu.SemaphoreType.DMA,
        ],
    )
    return f(memory, episode, memory_age, lru, maxage)


def kernel(episode, memory, memory_age, memory_usage, Wq, bq, Wk, bk, Wv, bv):
    lru, add1, maxage = _sc_topk(memory_age)

    usage2 = memory_usage.reshape(NBLK, 1, TM)
    add1_2 = add1.reshape(NBLK, 1, TM)
    attn, retrieved, usage_out = _attention(
        episode, memory, usage2, add1_2, Wq, bq, Wk, bk, Wv, bv)
    new_usage = usage_out.reshape(M)

    new_memory, new_age = _sc_apply(memory, episode, memory_age, lru, maxage)

    return (retrieved, attn, new_memory, new_age, new_usage)
